# baseline (speedup 1.0000x reference)
import numpy as np
import jax
import jax.numpy as jnp
from jax.experimental import pallas as pl


def _rot(s, x, y, rx, ry):
    if ry == 0:
        if rx == 1:
            x = s - 1 - x
            y = s - 1 - y
        return (y, x)
    return (x, y)


def _d2xy(n, d):
    x = y = 0
    t = d
    s = 1
    while s < n:
        rx = 1 & t // 2
        ry = 1 & (t ^ rx)
        x, y = _rot(s, x, y, rx, ry)
        x += s * rx
        y += s * ry
        t //= 4
        s *= 2
    return (x, y)


_N = 64
_HW = _N * _N
_CHUNK = 256
_SQ = 16
_NCHUNK = _HW // _CHUNK

_xy = np.array([_d2xy(_N, d) for d in range(_HW)])
_QX = (_xy[::_CHUNK, 0] // _SQ).astype(int)
_QY = (_xy[::_CHUNK, 1] // _SQ).astype(int)

_ORDER = np.argsort(_QY, kind="stable")
assert all(_QY[_ORDER[t]] == t // 4 for t in range(_NCHUNK))

_BAND = _SQ * _N
_P = np.zeros((_NCHUNK, _BAND, _CHUNK), np.float32)
for _t in range(_NCHUNK):
    _j = int(_ORDER[_t])
    for _dl in range(_CHUNK):
        _x, _y = _xy[_j * _CHUNK + _dl]
        _P[_t, (_y - _SQ * _QY[_j]) * _N + _x, _dl] = 1.0
_P_CONST = jnp.asarray(_P, dtype=jnp.bfloat16)
_ORDER_L = [int(v) for v in _ORDER]

_BB = 16


def _sel(j, table):
    r = jnp.int32(table[0])
    for k in range(1, len(table)):
        r = jnp.where(j == k, jnp.int32(table[k]), r)
    return r


def _hilbert_kernel(p_ref, x_ref, o_ref):
    t = pl.program_id(1)
    pt = p_ref[t]
    for bb in range(0, _BB, 4):
        xin = jnp.concatenate(
            [x_ref[bb + i] for i in range(4)], axis=0).astype(jnp.bfloat16)
        res = jax.lax.dot_general(
            pt, xin,
            (((0,), (1,)), ((), ())),
            preferred_element_type=jnp.float32,
        ).astype(o_ref.dtype)
        for i in range(4):
            o_ref[bb + i] = res[:, 128 * i:128 * (i + 1)]


def kernel(x):
    B, C, H, W = x.shape
    assert (H, W) == (_N, _N) and B % _BB == 0
    x3 = x.reshape(B, C, _HW)
    grid = (B // _BB, _NCHUNK)
    out = pl.pallas_call(
        _hilbert_kernel,
        grid=grid,
        in_specs=[
            pl.BlockSpec(_P_CONST.shape, lambda b, t: (0, 0, 0)),
            pl.BlockSpec((_BB, C, _BAND), lambda b, t: (b, 0, t // 4)),
        ],
        out_specs=pl.BlockSpec((_BB, _CHUNK, C),
                               lambda b, t: (b, _sel(t, _ORDER_L), 0)),
        out_shape=jax.ShapeDtypeStruct((B, _HW, C), x.dtype),
    )(_P_CONST, x3)
    return out


# speedup vs baseline: 1.0676x; 1.0676x over previous
import numpy as np
import jax
import jax.numpy as jnp
from jax.experimental import pallas as pl


def _rot(s, x, y, rx, ry):
    if ry == 0:
        if rx == 1:
            x = s - 1 - x
            y = s - 1 - y
        return (y, x)
    return (x, y)


def _d2xy(n, d):
    x = y = 0
    t = d
    s = 1
    while s < n:
        rx = 1 & t // 2
        ry = 1 & (t ^ rx)
        x, y = _rot(s, x, y, rx, ry)
        x += s * rx
        y += s * ry
        t //= 4
        s *= 2
    return (x, y)


_N = 64
_HW = _N * _N
_CHUNK = 256
_SQ = 16
_NCHUNK = _HW // _CHUNK

_xy = np.array([_d2xy(_N, d) for d in range(_HW)])
_QX = (_xy[::_CHUNK, 0] // _SQ).astype(int)
_QY = (_xy[::_CHUNK, 1] // _SQ).astype(int)

_ORDER = np.argsort(_QY, kind="stable")
assert all(_QY[_ORDER[t]] == t // 4 for t in range(_NCHUNK))

_BAND = _SQ * _N
_P = np.zeros((_NCHUNK, _BAND, _CHUNK), np.float32)
for _t in range(_NCHUNK):
    _j = int(_ORDER[_t])
    for _dl in range(_CHUNK):
        _x, _y = _xy[_j * _CHUNK + _dl]
        _P[_t, (_y - _SQ * _QY[_j]) * _N + _x, _dl] = 1.0
_P_CONST = jnp.asarray(_P, dtype=jnp.bfloat16)
_ORDER_L = [int(v) for v in _ORDER]

_BB = 32


def _sel(j, table):
    r = jnp.int32(table[0])
    for k in range(1, len(table)):
        r = jnp.where(j == k, jnp.int32(table[k]), r)
    return r


def _hilbert_kernel(p_ref, x_ref, o_ref):
    t = pl.program_id(1)
    pt = p_ref[t]
    for bb in range(0, _BB, 2):
        xin = jnp.concatenate(
            [x_ref[bb], x_ref[bb + 1]], axis=0).astype(jnp.bfloat16)
        res = jax.lax.dot_general(
            pt, xin,
            (((0,), (1,)), ((), ())),
            preferred_element_type=jnp.float32,
        ).astype(o_ref.dtype)
        o_ref[bb] = res[:, :128]
        o_ref[bb + 1] = res[:, 128:]


def kernel(x):
    B, C, H, W = x.shape
    assert (H, W) == (_N, _N) and B % _BB == 0
    x3 = x.reshape(B, C, _HW)
    grid = (B // _BB, _NCHUNK)
    out = pl.pallas_call(
        _hilbert_kernel,
        grid=grid,
        in_specs=[
            pl.BlockSpec(_P_CONST.shape, lambda b, t: (0, 0, 0)),
            pl.BlockSpec((_BB, C, _BAND), lambda b, t: (b, 0, t // 4)),
        ],
        out_specs=pl.BlockSpec((_BB, _CHUNK, C),
                               lambda b, t: (b, _sel(t, _ORDER_L), 0)),
        out_shape=jax.ShapeDtypeStruct((B, _HW, C), x.dtype),
    )(_P_CONST, x3)
    return out
